# baseline (speedup 1.0000x reference)
import jax
import jax.numpy as jnp
from jax.experimental import pallas as pl
from jax.experimental.pallas import tpu as pltpu

B, T, K = 512, 512, 128
START, STOP = 126, 127
NEG = -10000.0

B_BLK = 256
NB = B // B_BLK
T_BLK = 64
NT = T // T_BLK


def _crf_body(feats_ref, tags_ref, tr_ref, trT_ref, logz_ref, gold_ref,
              alpha_s, ohprev_s, emit_s, trans_s, ident_s):
    it = pl.program_id(1)
    ksub = jax.lax.broadcasted_iota(jnp.int32, (K, B_BLK), 0)

    @pl.when(it == 0)
    def _init():
        alpha_s[...] = jnp.where(ksub == START, 0.0, NEG)
        ohprev_s[...] = jnp.where(ksub == START, 1.0, 0.0)
        emit_s[...] = jnp.zeros((K, B_BLK), jnp.float32)
        trans_s[...] = jnp.zeros((K, B_BLK), jnp.float32)
        bsub = jax.lax.broadcasted_iota(jnp.int32, (B_BLK, B_BLK), 0)
        blane = jax.lax.broadcasted_iota(jnp.int32, (B_BLK, B_BLK), 1)
        ident_s[...] = jnp.where(bsub == blane, 1.0, 0.0)

    tr = tr_ref[...]
    trT = trT_ref[...]
    c = jnp.max(tr, axis=1, keepdims=True)
    et = jnp.exp(tr - c)

    ident = ident_s[...]

    def step(i, carry):
        feat_bk = feats_ref[:, pl.ds(i, 1), :].reshape(B_BLK, K)
        featT = jax.lax.dot_general(
            feat_bk, ident, (((0,), (0,)), ((), ())),
            preferred_element_type=jnp.float32)
        tagrow = tags_ref[i]
        ohT = jnp.where(ksub == tagrow, 1.0, 0.0)

        alpha = alpha_s[...]
        m = jnp.max(alpha, axis=0, keepdims=True)
        w = jnp.exp(alpha - m)
        s = jnp.dot(et, w, preferred_element_type=jnp.float32)
        alpha_s[...] = m + c + jnp.log(s) + featT

        rows = jnp.dot(trT, ohT, preferred_element_type=jnp.float32)
        trans_s[...] += rows * ohprev_s[...]
        emit_s[...] += featT * ohT
        ohprev_s[...] = ohT
        return carry

    jax.lax.fori_loop(0, T_BLK, step, 0)

    @pl.when(it == NT - 1)
    def _fin():
        alpha = alpha_s[...]
        stop_row = tr_ref[STOP:STOP + 1, :]
        c2 = jnp.max(stop_row, axis=1, keepdims=True)
        estop = jnp.exp(stop_row - c2)
        m2 = jnp.max(alpha, axis=0, keepdims=True)
        w2 = jnp.exp(alpha - m2)
        z = jnp.dot(estop, w2, preferred_element_type=jnp.float32)
        logz_ref[...] = (m2 + c2 + jnp.log(z)).reshape(1, 1, B_BLK)

        stopv = jnp.dot(stop_row, ohprev_s[...],
                        preferred_element_type=jnp.float32)
        gold = jnp.sum(emit_s[...] + trans_s[...], axis=0, keepdims=True) + stopv
        gold_ref[...] = gold.reshape(1, 1, B_BLK)


def kernel(feats, tags, lengths, transitions):
    del lengths
    tagsT = jnp.transpose(tags.astype(jnp.int32), (1, 0)).reshape(T, 1, B)
    tr = transitions.astype(jnp.float32)
    trT = tr.T

    grid = (NB, NT)
    logz, gold = pl.pallas_call(
        _crf_body,
        grid=grid,
        in_specs=[
            pl.BlockSpec((B_BLK, T_BLK, K), lambda ib, it: (ib, it, 0)),
            pl.BlockSpec((T_BLK, 1, B_BLK), lambda ib, it: (it, 0, ib)),
            pl.BlockSpec((K, K), lambda ib, it: (0, 0)),
            pl.BlockSpec((K, K), lambda ib, it: (0, 0)),
        ],
        out_specs=[
            pl.BlockSpec((1, 1, B_BLK), lambda ib, it: (ib, 0, 0)),
            pl.BlockSpec((1, 1, B_BLK), lambda ib, it: (ib, 0, 0)),
        ],
        out_shape=[
            jax.ShapeDtypeStruct((NB, 1, B_BLK), jnp.float32),
            jax.ShapeDtypeStruct((NB, 1, B_BLK), jnp.float32),
        ],
        scratch_shapes=[pltpu.VMEM((K, B_BLK), jnp.float32)] * 4
        + [pltpu.VMEM((B_BLK, B_BLK), jnp.float32)],
        compiler_params=pltpu.CompilerParams(
            dimension_semantics=("parallel", "arbitrary"),
        ),
    )(feats, tagsT, tr, trT)
    return jnp.sum(logz) - jnp.sum(gold)


# speedup vs baseline: 1.5551x; 1.5551x over previous
import jax
import jax.numpy as jnp
from jax.experimental import pallas as pl
from jax.experimental.pallas import tpu as pltpu

B, T, K = 512, 512, 128
START, STOP = 126, 127
NEG = -10000.0

B_BLK = 256
NB = B // B_BLK
T_BLK = 64
NT = T // T_BLK


def _crf_body(feats_ref, tags_ref, tr_ref, trT_ref, logz_ref, gold_ref,
              alpha_s, ohprev_s, emit_s, trans_s, ident_s, featT_s):
    it = pl.program_id(1)
    ksub = jax.lax.broadcasted_iota(jnp.int32, (K, B_BLK), 0)

    @pl.when(it == 0)
    def _init():
        alpha_s[...] = jnp.where(ksub == START, 0.0, NEG)
        ohprev_s[...] = jnp.where(ksub == START, 1.0, 0.0)
        emit_s[...] = jnp.zeros((K, B_BLK), jnp.float32)
        trans_s[...] = jnp.zeros((K, B_BLK), jnp.float32)
        bsub = jax.lax.broadcasted_iota(jnp.int32, (B_BLK, B_BLK), 0)
        blane = jax.lax.broadcasted_iota(jnp.int32, (B_BLK, B_BLK), 1)
        ident_s[...] = jnp.where(bsub == blane, 1.0, 0.0)

    tr = tr_ref[...]
    trT = trT_ref[...]
    c = jnp.max(tr, axis=1, keepdims=True)
    et = jnp.exp(tr - c)

    ident = ident_s[...]

    def chunk(j, carry):
        base = pl.multiple_of(j * 8, 8)
        fc = feats_ref[:, pl.ds(base, 8), :]
        featT_s[...] = jax.lax.dot_general(
            fc, ident, (((0,), (0,)), ((), ())),
            preferred_element_type=jnp.float32)

        for s in range(8):
            featT = featT_s[s]
            tagrow = tags_ref[base + s]
            ohT = jnp.where(ksub == tagrow, 1.0, 0.0)

            alpha = alpha_s[...]
            m = jnp.max(alpha, axis=0, keepdims=True)
            w = jnp.exp(alpha - m)
            sdot = jnp.dot(et, w, preferred_element_type=jnp.float32)
            alpha_s[...] = m + c + jnp.log(sdot) + featT

            rows = jnp.dot(trT, ohT, preferred_element_type=jnp.float32)
            trans_s[...] += rows * ohprev_s[...]
            emit_s[...] += featT * ohT
            ohprev_s[...] = ohT
        return carry

    jax.lax.fori_loop(0, T_BLK // 8, chunk, 0)

    @pl.when(it == NT - 1)
    def _fin():
        alpha = alpha_s[...]
        stop_row = tr_ref[STOP:STOP + 1, :]
        c2 = jnp.max(stop_row, axis=1, keepdims=True)
        estop = jnp.exp(stop_row - c2)
        m2 = jnp.max(alpha, axis=0, keepdims=True)
        w2 = jnp.exp(alpha - m2)
        z = jnp.dot(estop, w2, preferred_element_type=jnp.float32)
        logz_ref[...] = (m2 + c2 + jnp.log(z)).reshape(1, 1, B_BLK)

        stopv = jnp.dot(stop_row, ohprev_s[...],
                        preferred_element_type=jnp.float32)
        gold = jnp.sum(emit_s[...] + trans_s[...], axis=0, keepdims=True) + stopv
        gold_ref[...] = gold.reshape(1, 1, B_BLK)


def kernel(feats, tags, lengths, transitions):
    del lengths
    tagsT = jnp.transpose(tags.astype(jnp.int32), (1, 0)).reshape(T, 1, B)
    tr = transitions.astype(jnp.float32)
    trT = tr.T

    grid = (NB, NT)
    logz, gold = pl.pallas_call(
        _crf_body,
        grid=grid,
        in_specs=[
            pl.BlockSpec((B_BLK, T_BLK, K), lambda ib, it: (ib, it, 0)),
            pl.BlockSpec((T_BLK, 1, B_BLK), lambda ib, it: (it, 0, ib)),
            pl.BlockSpec((K, K), lambda ib, it: (0, 0)),
            pl.BlockSpec((K, K), lambda ib, it: (0, 0)),
        ],
        out_specs=[
            pl.BlockSpec((1, 1, B_BLK), lambda ib, it: (ib, 0, 0)),
            pl.BlockSpec((1, 1, B_BLK), lambda ib, it: (ib, 0, 0)),
        ],
        out_shape=[
            jax.ShapeDtypeStruct((NB, 1, B_BLK), jnp.float32),
            jax.ShapeDtypeStruct((NB, 1, B_BLK), jnp.float32),
        ],
        scratch_shapes=[pltpu.VMEM((K, B_BLK), jnp.float32)] * 4
        + [pltpu.VMEM((B_BLK, B_BLK), jnp.float32),
           pltpu.VMEM((8, K, B_BLK), jnp.float32)],
        compiler_params=pltpu.CompilerParams(
            dimension_semantics=("parallel", "arbitrary"),
        ),
    )(feats, tagsT, tr, trT)
    return jnp.sum(logz) - jnp.sum(gold)


# speedup vs baseline: 1.6685x; 1.0729x over previous
import jax
import jax.numpy as jnp
from jax.experimental import pallas as pl
from jax.experimental.pallas import tpu as pltpu

B, T, K = 512, 512, 128
START, STOP = 126, 127
NEG = -10000.0

B_BLK = 256
NB = B // B_BLK
T_BLK = 64
NT = T // T_BLK
G = 2
BG = B_BLK // G


def _crf_body(feats_ref, tags_ref, tr_ref, trT_ref, logz_ref, gold_ref,
              alpha_s, ohprev_s, acc_s, ident_s, featT_s):
    it = pl.program_id(1)
    ksub = jax.lax.broadcasted_iota(jnp.int32, (K, B_BLK), 0)

    @pl.when(it == 0)
    def _init():
        alpha_s[...] = jnp.where(ksub == START, 0.0, NEG)
        ohprev_s[...] = jnp.where(ksub == START, 1.0, 0.0)
        acc_s[...] = jnp.zeros((K, B_BLK), jnp.float32)
        bsub = jax.lax.broadcasted_iota(jnp.int32, (B_BLK, B_BLK), 0)
        blane = jax.lax.broadcasted_iota(jnp.int32, (B_BLK, B_BLK), 1)
        ident_s[...] = jnp.where(bsub == blane, 1.0, 0.0)

    tr = tr_ref[...]
    trT = trT_ref[...]
    c = jnp.max(tr, axis=1, keepdims=True)
    et = jnp.exp(tr - c)
    ident = ident_s[...]
    ksub_g = ksub[:, :BG]

    def chunk(j, carry):
        alpha, ohprev, acc = carry
        base = pl.multiple_of(j * 8, 8)
        fc = feats_ref[:, pl.ds(base, 8), :]
        featT_s[...] = jax.lax.dot_general(
            fc, ident, (((0,), (0,)), ((), ())),
            preferred_element_type=jnp.float32)

        for s in range(8):
            tagrow = tags_ref[base + s]
            alpha_n, ohprev_n, acc_n = [], [], []
            for g in range(G):
                lo, hi = g * BG, (g + 1) * BG
                featT = featT_s[s, :, lo:hi]
                ohT = jnp.where(ksub_g == tagrow[:, lo:hi], 1.0, 0.0)

                a = alpha[g]
                m = jnp.max(a, axis=0, keepdims=True)
                w = jnp.exp(a - m)
                sdot = jnp.dot(et, w, preferred_element_type=jnp.float32)
                alpha_n.append(m + c + jnp.log(sdot) + featT)

                rows = jnp.dot(trT, ohT, preferred_element_type=jnp.float32)
                acc_n.append(acc[g] + featT * ohT + rows * ohprev[g])
                ohprev_n.append(ohT)
            alpha, ohprev, acc = tuple(alpha_n), tuple(ohprev_n), tuple(acc_n)
        return alpha, ohprev, acc

    def split(ref):
        v = ref[...]
        return tuple(v[:, g * BG:(g + 1) * BG] for g in range(G))

    carry0 = (split(alpha_s), split(ohprev_s), split(acc_s))
    alpha, ohprev, acc = jax.lax.fori_loop(0, T_BLK // 8, chunk, carry0)
    alpha_s[...] = jnp.concatenate(alpha, axis=1)
    ohprev_s[...] = jnp.concatenate(ohprev, axis=1)
    acc_s[...] = jnp.concatenate(acc, axis=1)

    @pl.when(it == NT - 1)
    def _fin():
        alpha = alpha_s[...]
        stop_row = tr_ref[STOP:STOP + 1, :]
        c2 = jnp.max(stop_row, axis=1, keepdims=True)
        estop = jnp.exp(stop_row - c2)
        m2 = jnp.max(alpha, axis=0, keepdims=True)
        w2 = jnp.exp(alpha - m2)
        z = jnp.dot(estop, w2, preferred_element_type=jnp.float32)
        logz_ref[...] = (m2 + c2 + jnp.log(z)).reshape(1, 1, B_BLK)

        stopv = jnp.dot(stop_row, ohprev_s[...],
                        preferred_element_type=jnp.float32)
        gold = jnp.sum(acc_s[...], axis=0, keepdims=True) + stopv
        gold_ref[...] = gold.reshape(1, 1, B_BLK)


def kernel(feats, tags, lengths, transitions):
    del lengths
    tagsT = jnp.transpose(tags.astype(jnp.int32), (1, 0)).reshape(T, 1, B)
    tr = transitions.astype(jnp.float32)
    trT = tr.T

    grid = (NB, NT)
    logz, gold = pl.pallas_call(
        _crf_body,
        grid=grid,
        in_specs=[
            pl.BlockSpec((B_BLK, T_BLK, K), lambda ib, it: (ib, it, 0)),
            pl.BlockSpec((T_BLK, 1, B_BLK), lambda ib, it: (it, 0, ib)),
            pl.BlockSpec((K, K), lambda ib, it: (0, 0)),
            pl.BlockSpec((K, K), lambda ib, it: (0, 0)),
        ],
        out_specs=[
            pl.BlockSpec((1, 1, B_BLK), lambda ib, it: (ib, 0, 0)),
            pl.BlockSpec((1, 1, B_BLK), lambda ib, it: (ib, 0, 0)),
        ],
        out_shape=[
            jax.ShapeDtypeStruct((NB, 1, B_BLK), jnp.float32),
            jax.ShapeDtypeStruct((NB, 1, B_BLK), jnp.float32),
        ],
        scratch_shapes=[pltpu.VMEM((K, B_BLK), jnp.float32)] * 3
        + [pltpu.VMEM((B_BLK, B_BLK), jnp.float32),
           pltpu.VMEM((8, K, B_BLK), jnp.float32)],
        compiler_params=pltpu.CompilerParams(
            dimension_semantics=("parallel", "arbitrary"),
        ),
    )(feats, tagsT, tr, trT)
    return jnp.sum(logz) - jnp.sum(gold)


# speedup vs baseline: 1.9177x; 1.1494x over previous
import jax
import jax.numpy as jnp
from jax.experimental import pallas as pl
from jax.experimental.pallas import tpu as pltpu

B, T, K = 512, 512, 128
START, STOP = 126, 127
NEG = -10000.0

B_BLK = 256
NB = B // B_BLK
T_BLK = 64
NT = T // T_BLK
G = 2
BG = B_BLK // G


def _crf_body(feats_ref, tags_ref, tr_ref, trT_ref, logz_ref, gold_ref,
              a_s, logacc_s, ohprev_s, acc_s, ident_s, featT_s):
    it = pl.program_id(1)
    ksub = jax.lax.broadcasted_iota(jnp.int32, (K, B_BLK), 0)

    @pl.when(it == 0)
    def _init():
        a_s[...] = jnp.where(ksub == START, 1.0, 0.0)
        ohprev_s[...] = jnp.where(ksub == START, 1.0, 0.0)
        acc_s[...] = jnp.zeros((K, B_BLK), jnp.float32)
        logacc_s[...] = jnp.zeros((1, B_BLK), jnp.float32)
        bsub = jax.lax.broadcasted_iota(jnp.int32, (B_BLK, B_BLK), 0)
        blane = jax.lax.broadcasted_iota(jnp.int32, (B_BLK, B_BLK), 1)
        ident_s[...] = jnp.where(bsub == blane, 1.0, 0.0)

    tr = tr_ref[...]
    trT = trT_ref[...]
    tmax = jnp.max(jnp.max(tr, axis=1, keepdims=True), axis=0, keepdims=True)
    etc = jnp.exp(tr - tmax)
    ident = ident_s[...]
    ksub_g = ksub[:, :BG]

    def chunk(j, carry):
        a, logacc, ohprev, acc = carry
        base = pl.multiple_of(j * 8, 8)
        fc = feats_ref[:, pl.ds(base, 8), :]
        featT_s[...] = jax.lax.dot_general(
            fc, ident, (((0,), (0,)), ((), ())),
            preferred_element_type=jnp.float32)

        for s in range(8):
            tagrow = tags_ref[base + s]
            a_n, la_n, ohprev_n, acc_n = [], [], [], []
            for g in range(G):
                lo, hi = g * BG, (g + 1) * BG
                featT = featT_s[s, :, lo:hi]
                ohT = jnp.where(ksub_g == tagrow[:, lo:hi], 1.0, 0.0)

                z = jnp.dot(etc, a[g], preferred_element_type=jnp.float32)
                an = z * jnp.exp(featT)
                la = logacc[g]
                if s % 4 == 3:
                    m4 = jnp.max(an, axis=0, keepdims=True)
                    la = la + jnp.log(m4)
                    an = an * (1.0 / m4)
                a_n.append(an)
                la_n.append(la)

                rows = jnp.dot(trT, ohT, preferred_element_type=jnp.float32)
                acc_n.append(acc[g] + featT * ohT + rows * ohprev[g])
                ohprev_n.append(ohT)
            a, logacc, ohprev, acc = (tuple(a_n), tuple(la_n),
                                      tuple(ohprev_n), tuple(acc_n))
        return a, logacc, ohprev, acc

    def split(ref):
        v = ref[...]
        return tuple(v[:, g * BG:(g + 1) * BG] for g in range(G))

    carry0 = (split(a_s), split(logacc_s), split(ohprev_s), split(acc_s))
    a, logacc, ohprev, acc = jax.lax.fori_loop(0, T_BLK // 8, chunk, carry0)
    a_s[...] = jnp.concatenate(a, axis=1)
    logacc_s[...] = jnp.concatenate(logacc, axis=1)
    ohprev_s[...] = jnp.concatenate(ohprev, axis=1)
    acc_s[...] = jnp.concatenate(acc, axis=1)

    @pl.when(it == NT - 1)
    def _fin():
        av = a_s[...]
        stop_row = tr_ref[STOP:STOP + 1, :]
        c2 = jnp.max(stop_row, axis=1, keepdims=True)
        estop = jnp.exp(stop_row - c2)
        z = jnp.dot(estop, av, preferred_element_type=jnp.float32)
        logz = logacc_s[...] + c2 + jnp.log(z) + jnp.float32(T) * tmax
        logz_ref[...] = logz.reshape(1, 1, B_BLK)

        stopv = jnp.dot(stop_row, ohprev_s[...],
                        preferred_element_type=jnp.float32)
        gold = jnp.sum(acc_s[...], axis=0, keepdims=True) + stopv
        gold_ref[...] = gold.reshape(1, 1, B_BLK)


def kernel(feats, tags, lengths, transitions):
    del lengths
    tagsT = jnp.transpose(tags.astype(jnp.int32), (1, 0)).reshape(T, 1, B)
    tr = transitions.astype(jnp.float32)
    trT = tr.T

    grid = (NB, NT)
    logz, gold = pl.pallas_call(
        _crf_body,
        grid=grid,
        in_specs=[
            pl.BlockSpec((B_BLK, T_BLK, K), lambda ib, it: (ib, it, 0)),
            pl.BlockSpec((T_BLK, 1, B_BLK), lambda ib, it: (it, 0, ib)),
            pl.BlockSpec((K, K), lambda ib, it: (0, 0)),
            pl.BlockSpec((K, K), lambda ib, it: (0, 0)),
        ],
        out_specs=[
            pl.BlockSpec((1, 1, B_BLK), lambda ib, it: (ib, 0, 0)),
            pl.BlockSpec((1, 1, B_BLK), lambda ib, it: (ib, 0, 0)),
        ],
        out_shape=[
            jax.ShapeDtypeStruct((NB, 1, B_BLK), jnp.float32),
            jax.ShapeDtypeStruct((NB, 1, B_BLK), jnp.float32),
        ],
        scratch_shapes=[pltpu.VMEM((K, B_BLK), jnp.float32),
                        pltpu.VMEM((1, B_BLK), jnp.float32),
                        pltpu.VMEM((K, B_BLK), jnp.float32),
                        pltpu.VMEM((K, B_BLK), jnp.float32),
                        pltpu.VMEM((B_BLK, B_BLK), jnp.float32),
                        pltpu.VMEM((8, K, B_BLK), jnp.float32)],
        compiler_params=pltpu.CompilerParams(
            dimension_semantics=("parallel", "arbitrary"),
        ),
    )(feats, tagsT, tr, trT)
    return jnp.sum(logz) - jnp.sum(gold)


# speedup vs baseline: 2.0220x; 1.0543x over previous
import jax
import jax.numpy as jnp
from jax.experimental import pallas as pl
from jax.experimental.pallas import tpu as pltpu

B, T, K = 512, 512, 128
START, STOP = 126, 127
NEG = -10000.0

B_BLK = 256
NB = B // B_BLK
T_BLK = 64
NT = T // T_BLK
G = 2
BG = B_BLK // G


def _crf_body(feats_ref, tags_ref, tr_ref, trT_ref, logz_ref, gold_ref,
              a_s, logacc_s, ohprev_s, acc_s, ident_s, featT_s):
    it = pl.program_id(1)
    ksub = jax.lax.broadcasted_iota(jnp.int32, (K, B_BLK), 0)

    @pl.when(it == 0)
    def _init():
        a_s[...] = jnp.where(ksub == START, 1.0, 0.0)
        ohprev_s[...] = jnp.where(ksub == START, 1.0, 0.0)
        acc_s[...] = jnp.zeros((K, B_BLK), jnp.float32)
        logacc_s[...] = jnp.zeros((1, B_BLK), jnp.float32)
        bsub = jax.lax.broadcasted_iota(jnp.int32, (B_BLK, B_BLK), 0)
        blane = jax.lax.broadcasted_iota(jnp.int32, (B_BLK, B_BLK), 1)
        ident_s[...] = jnp.where(bsub == blane, 1.0, 0.0)

    tr = tr_ref[...]
    trT = trT_ref[...]
    tmax = jnp.max(jnp.max(tr, axis=1, keepdims=True), axis=0, keepdims=True)
    etc = jnp.exp(tr - tmax)
    ident = ident_s[...]
    ksub_g = ksub[:, :BG]

    def half(jj, slot, a, logacc):
        base = pl.multiple_of(jj * 8, 8)
        fc = feats_ref[:, pl.ds(base, 8), :]
        featT_s[slot] = jax.lax.dot_general(
            fc, ident, (((0,), (0,)), ((), ())),
            preferred_element_type=jnp.float32)

        for s in range(8):
            tagrow = tags_ref[base + s]
            a_n, la_n = [], []
            for g in range(G):
                lo, hi = g * BG, (g + 1) * BG
                featT = featT_s[slot, s, :, lo:hi]
                ohT = jnp.where(ksub_g == tagrow[:, lo:hi], 1.0, 0.0)

                z = jnp.dot(etc, a[g], preferred_element_type=jnp.float32)
                an = z * jnp.exp(featT)
                la = logacc[g]
                if s % 4 == 3:
                    m4 = jnp.max(an, axis=0, keepdims=True)
                    la = la + jnp.log(m4)
                    an = an * (1.0 / m4)
                a_n.append(an)
                la_n.append(la)

                rows = jnp.dot(trT, ohT, preferred_element_type=jnp.float32)
                acc_s[:, lo:hi] += featT * ohT + rows * ohprev_s[:, lo:hi]
                ohprev_s[:, lo:hi] = ohT
            a, logacc = tuple(a_n), tuple(la_n)
        return a, logacc

    def chunk2(j, carry):
        a, logacc = carry
        a, logacc = half(j * 2, 0, a, logacc)
        a, logacc = half(j * 2 + 1, 1, a, logacc)
        return a, logacc

    def split(ref):
        v = ref[...]
        return tuple(v[:, g * BG:(g + 1) * BG] for g in range(G))

    carry0 = (split(a_s), split(logacc_s))
    a, logacc = jax.lax.fori_loop(0, T_BLK // 16, chunk2, carry0)
    a_s[...] = jnp.concatenate(a, axis=1)
    logacc_s[...] = jnp.concatenate(logacc, axis=1)

    @pl.when(it == NT - 1)
    def _fin():
        av = a_s[...]
        stop_row = tr_ref[STOP:STOP + 1, :]
        c2 = jnp.max(stop_row, axis=1, keepdims=True)
        estop = jnp.exp(stop_row - c2)
        z = jnp.dot(estop, av, preferred_element_type=jnp.float32)
        logz = logacc_s[...] + c2 + jnp.log(z) + jnp.float32(T) * tmax
        logz_ref[...] = logz.reshape(1, 1, B_BLK)

        stopv = jnp.dot(stop_row, ohprev_s[...],
                        preferred_element_type=jnp.float32)
        gold = jnp.sum(acc_s[...], axis=0, keepdims=True) + stopv
        gold_ref[...] = gold.reshape(1, 1, B_BLK)


def kernel(feats, tags, lengths, transitions):
    del lengths
    tagsT = jnp.transpose(tags.astype(jnp.int32), (1, 0)).reshape(T, 1, B)
    tr = transitions.astype(jnp.float32)
    trT = tr.T

    grid = (NB, NT)
    logz, gold = pl.pallas_call(
        _crf_body,
        grid=grid,
        in_specs=[
            pl.BlockSpec((B_BLK, T_BLK, K), lambda ib, it: (ib, it, 0)),
            pl.BlockSpec((T_BLK, 1, B_BLK), lambda ib, it: (it, 0, ib)),
            pl.BlockSpec((K, K), lambda ib, it: (0, 0)),
            pl.BlockSpec((K, K), lambda ib, it: (0, 0)),
        ],
        out_specs=[
            pl.BlockSpec((1, 1, B_BLK), lambda ib, it: (ib, 0, 0)),
            pl.BlockSpec((1, 1, B_BLK), lambda ib, it: (ib, 0, 0)),
        ],
        out_shape=[
            jax.ShapeDtypeStruct((NB, 1, B_BLK), jnp.float32),
            jax.ShapeDtypeStruct((NB, 1, B_BLK), jnp.float32),
        ],
        scratch_shapes=[pltpu.VMEM((K, B_BLK), jnp.float32),
                        pltpu.VMEM((1, B_BLK), jnp.float32),
                        pltpu.VMEM((K, B_BLK), jnp.float32),
                        pltpu.VMEM((K, B_BLK), jnp.float32),
                        pltpu.VMEM((B_BLK, B_BLK), jnp.float32),
                        pltpu.VMEM((2, 8, K, B_BLK), jnp.float32)],
        compiler_params=pltpu.CompilerParams(
            dimension_semantics=("parallel", "arbitrary"),
        ),
    )(feats, tagsT, tr, trT)
    return jnp.sum(logz) - jnp.sum(gold)
